# baseline (speedup 1.0000x reference)
import functools

import jax
import jax.numpy as jnp
from jax import lax
from jax.experimental import pallas as pl
from jax.experimental.pallas import tpu as pltpu
from jax.experimental.pallas import tpu_sc as plsc

NUM_CORES = 2
NUM_SUBCORES = 16
NUM_WORKERS = NUM_CORES * NUM_SUBCORES
LANES = 16

BATCH = 16384
DIM = 64
ENT = 1000000
NUM_REL = 1000

PART_SHIFT = 15
PART = 1 << PART_SHIFT
CH_SHIFT = 8
CH = 1 << CH_SHIFT
CAP = 1024
LIST = CAP + LANES
E1_PIECE = 2048
TAIL0 = (ENT // 128) * 128
TAILN = ENT - TAIL0

_i32 = jnp.int32


def _pc(mask):
    n = plsc.all_reduce_population_count(mask)
    n = jnp.asarray(n)
    return n[0] if n.ndim else n


def _body(e1_ref, r_ref, ent_t_ref, rel_t_ref, tail_t_ref, out_ref,
          e1buf, r_v, pids, ppos, cloc, cpos, chunk3, rel64, tail64, rg,
          csem0, csem1, osem, ssem):
    wid = lax.axis_index("s") * NUM_CORES + lax.axis_index("c")
    part_base = wid * PART
    span = jnp.maximum(0, jnp.minimum(PART, TAIL0 - part_base))
    nch_full = span >> CH_SHIFT

    iota = lax.iota(_i32, LANES)

    pltpu.sync_copy(r_ref, r_v)
    pltpu.sync_copy(rel_t_ref, rel64)
    pltpu.sync_copy(tail_t_ref, tail64)

    def scan_round(rnd):
        lo = rnd * CAP
        hi = lo + CAP

        def piece(p, carry):
            cnt, app = carry
            pltpu.sync_copy(e1_ref.at[pl.ds(p * E1_PIECE, E1_PIECE)], e1buf)

            def step(t, carry2):
                cnt2, app2 = carry2
                ev = e1buf[pl.ds(t * LANES, LANES)]
                m = (ev >> PART_SHIFT) == wid
                mi = m.astype(_i32)
                excl = plsc.cumsum(mi) - mi
                rank = cnt2 + excl
                sel = m & (rank >= lo) & (rank < hi)
                plsc.store_compressed(pids.at[pl.ds(app2, LANES)],
                                      ev - part_base, mask=sel)
                posv = iota + (p * E1_PIECE + t * LANES)
                plsc.store_compressed(ppos.at[pl.ds(app2, LANES)], posv,
                                      mask=sel)
                return cnt2 + _pc(m), app2 + _pc(sel)

            return lax.fori_loop(0, E1_PIECE // LANES, step, (cnt, app))

        cnt, app = lax.fori_loop(0, BATCH // E1_PIECE, piece,
                                 (jnp.asarray(0, _i32), jnp.asarray(0, _i32)))
        pids[pl.ds(app, LANES)] = jnp.full((LANES,), -1, _i32)
        return cnt, app

    def issue_chunk(ch, par_buf, sem):
        col0 = pl.multiple_of(part_base + ch * CH, CH)
        pltpu.make_async_copy(ent_t_ref.at[:, pl.ds(col0, CH)],
                              chunk3.at[par_buf], sem).start()

    def wait_chunk(par_buf, sem):
        pltpu.make_async_copy(ent_t_ref.at[:, pl.ds(0, CH)],
                              chunk3.at[par_buf], sem).wait()

    def process_members(app, member_fn, gather_fn):
        def ext(t, ccnt):
            base = t * LANES
            loc16 = pids[pl.ds(base, LANES)]
            pos16 = ppos[pl.ds(base, LANES)]
            m, lvec16 = member_fn(loc16)
            plsc.store_compressed(cloc.at[pl.ds(ccnt, LANES)], lvec16, mask=m)
            plsc.store_compressed(cpos.at[pl.ds(ccnt, LANES)], pos16, mask=m)
            return ccnt + _pc(m)

        titers = (app + LANES - 1) >> 4
        ccnt = lax.fori_loop(0, titers, ext, jnp.asarray(0, _i32))

        def group(g, carry):
            gbase = g * LANES
            lvec = cloc[pl.ds(gbase, LANES)]
            pvec = cpos[pl.ds(gbase, LANES)]
            gvalid = iota < (ccnt - gbase)
            gcnt = jnp.minimum(LANES, ccnt - gbase)
            rvec = plsc.load_gather(r_v, [pvec], mask=gvalid)
            iota_d = iota * DIM

            def dloop(d4, carry2):
                for k in range(4):
                    d = d4 * 4 + k
                    dv = jnp.full((LANES,), 0, _i32) + d
                    ent = gather_fn(dv, lvec, gvalid)
                    rel = plsc.load_gather(rel64, [dv, rvec], mask=gvalid)
                    plsc.store_scatter(rg, [iota_d + d], ent + rel,
                                       mask=gvalid)
                return carry2

            lax.fori_loop(0, DIM // 4, dloop, 0)

            for j in range(LANES):
                pj = pvec[j]

                @pl.when(j < gcnt)
                def _():
                    pltpu.make_async_copy(
                        rg.at[pl.ds(j * DIM, DIM)],
                        out_ref.at[pl.ds(pj * DIM, DIM)], osem).start()

            def drain(j, carry2):
                pltpu.make_async_copy(rg.at[pl.ds(0, DIM)],
                                      out_ref.at[pl.ds(0, DIM)], osem).wait()
                return carry2

            lax.fori_loop(0, gcnt, drain, 0)
            return carry

        ngr = (ccnt + LANES - 1) >> 4
        lax.fori_loop(0, ngr, group, 0)

    def chunk_member(ch):
        def fn(loc16):
            return (loc16 >> CH_SHIFT) == ch, loc16 & (CH - 1)
        return fn

    def tail_member(loc16):
        return (loc16 >> CH_SHIFT) == nch_full, loc16 & (CH - 1)

    def chunk_gather(par):
        parv = jnp.full((LANES,), 0, _i32) + par

        def fn(dv, lvec, gvalid):
            return plsc.load_gather(chunk3, [parv, dv, lvec], mask=gvalid)
        return fn

    def tail_gather(dv, lvec, gvalid):
        return plsc.load_gather(tail64, [dv, lvec], mask=gvalid)

    def sweep(app):
        @pl.when(nch_full > 0)
        def _():
            issue_chunk(jnp.asarray(0, _i32), 0, csem0)

        def pair(cc, carry):
            ch0 = cc * 2
            ch1 = ch0 + 1

            @pl.when(ch1 < nch_full)
            def _():
                issue_chunk(ch1, 1, csem1)

            wait_chunk(0, csem0)
            process_members(app, chunk_member(ch0), chunk_gather(0))

            @pl.when(ch1 < nch_full)
            def _():
                @pl.when(ch1 + 1 < nch_full)
                def _():
                    issue_chunk(ch1 + 1, 0, csem0)

                wait_chunk(1, csem1)
                process_members(app, chunk_member(ch1), chunk_gather(1))

            return carry

        lax.fori_loop(0, (nch_full + 1) >> 1, pair, 0)

        process_members(app, tail_member, tail_gather)

    total0, app0 = scan_round(jnp.asarray(0, _i32))
    sweep(app0)

    def more(carry):
        rnd, total = carry
        return (rnd * CAP) < total

    def round_body(carry):
        rnd, total = carry
        _, app = scan_round(rnd)
        sweep(app)
        return rnd + 1, total

    lax.while_loop(more, round_body, (jnp.asarray(1, _i32), total0))


@jax.jit
def _transe(e1_1d, r_1d, ent_t, rel_t, tail_t):
    mesh = plsc.VectorSubcoreMesh(core_axis_name="c", subcore_axis_name="s")
    kern = pl.kernel(
        _body,
        mesh=mesh,
        compiler_params=pltpu.CompilerParams(needs_layout_passes=False),
        out_type=jax.ShapeDtypeStruct((BATCH * DIM,), jnp.float32),
        scratch_types=[
            pltpu.VMEM((E1_PIECE,), _i32),
            pltpu.VMEM((BATCH,), _i32),
            pltpu.VMEM((LIST,), _i32),
            pltpu.VMEM((LIST,), _i32),
            pltpu.VMEM((LIST,), _i32),
            pltpu.VMEM((LIST,), _i32),
            pltpu.VMEM((2, DIM, CH), jnp.float32),
            pltpu.VMEM((DIM, NUM_REL), jnp.float32),
            pltpu.VMEM((DIM, TAILN), jnp.float32),
            pltpu.VMEM((LANES * DIM,), jnp.float32),
            pltpu.SemaphoreType.DMA,
            pltpu.SemaphoreType.DMA,
            pltpu.SemaphoreType.DMA,
            pltpu.SemaphoreType.DMA,
        ],
    )
    return kern(e1_1d, r_1d, ent_t, rel_t, tail_t)


def kernel(e1, r, entity_table, relation_table):
    out = _transe(e1, r, entity_table.T, relation_table.T,
                  entity_table[TAIL0:].T)
    return out.reshape(BATCH, DIM)


# speedup vs baseline: 1.0037x; 1.0037x over previous
import functools

import jax
import jax.numpy as jnp
from jax import lax
from jax.experimental import pallas as pl
from jax.experimental.pallas import tpu as pltpu
from jax.experimental.pallas import tpu_sc as plsc

NUM_CORES = 2
NUM_SUBCORES = 16
NUM_WORKERS = NUM_CORES * NUM_SUBCORES
LANES = 16

BATCH = 16384
DIM = 64
ENT = 1000000
NUM_REL = 1000

PART_SHIFT = 15
PART = 1 << PART_SHIFT
CH_SHIFT = 8
CH = 1 << CH_SHIFT
CAP = 1024
LIST = CAP + LANES
E1_PIECE = 2048
TAIL0 = (ENT // 128) * 128
TAILN = ENT - TAIL0

_i32 = jnp.int32


def _pc(mask):
    n = plsc.all_reduce_population_count(mask)
    n = jnp.asarray(n)
    return n[0] if n.ndim else n


def _body(e1_ref, r_ref, ent_t_ref, rel_t_ref, tail_t_ref, out_ref,
          e1buf, r_v, pids, ppos, cloc, cpos, chunk3, rel64, tail64, rg,
          csem0, csem1, osem, ssem):
    wid = lax.axis_index("s") * NUM_CORES + lax.axis_index("c")
    part_base = wid * PART
    span = jnp.maximum(0, jnp.minimum(PART, TAIL0 - part_base))
    nch_full = span >> CH_SHIFT

    iota = lax.iota(_i32, LANES)

    pltpu.sync_copy(r_ref, r_v)
    pltpu.sync_copy(rel_t_ref, rel64)
    pltpu.sync_copy(tail_t_ref, tail64)

    def scan_round(rnd):
        lo = rnd * CAP
        hi = lo + CAP

        def piece(p, carry):
            cnt, app = carry
            pltpu.sync_copy(e1_ref.at[pl.ds(p * E1_PIECE, E1_PIECE)], e1buf)

            def step(t, carry2):
                cnt2, app2 = carry2
                ev = e1buf[pl.ds(t * LANES, LANES)]
                m = (ev >> PART_SHIFT) == wid
                mi = m.astype(_i32)
                excl = plsc.cumsum(mi) - mi
                rank = cnt2 + excl
                sel = m & (rank >= lo) & (rank < hi)
                plsc.store_compressed(pids.at[pl.ds(app2, LANES)],
                                      ev - part_base, mask=sel)
                posv = iota + (p * E1_PIECE + t * LANES)
                plsc.store_compressed(ppos.at[pl.ds(app2, LANES)], posv,
                                      mask=sel)
                return cnt2 + _pc(m), app2 + _pc(sel)

            return lax.fori_loop(0, E1_PIECE // LANES, step, (cnt, app))

        cnt, app = lax.fori_loop(0, BATCH // E1_PIECE, piece,
                                 (jnp.asarray(0, _i32), jnp.asarray(0, _i32)))
        pids[pl.ds(app, LANES)] = jnp.full((LANES,), -1, _i32)
        return cnt, app

    def issue_chunk(ch, par_buf, sem):
        col0 = pl.multiple_of(part_base + ch * CH, CH)
        pltpu.make_async_copy(ent_t_ref.at[:, pl.ds(col0, CH)],
                              chunk3.at[par_buf], sem).start()

    def wait_chunk(par_buf, sem):
        pltpu.make_async_copy(ent_t_ref.at[:, pl.ds(0, CH)],
                              chunk3.at[par_buf], sem).wait()

    def process_members(app, member_fn, gather_fn):
        def ext(t, ccnt):
            base = t * LANES
            loc16 = pids[pl.ds(base, LANES)]
            pos16 = ppos[pl.ds(base, LANES)]
            m, lvec16 = member_fn(loc16)
            plsc.store_compressed(cloc.at[pl.ds(ccnt, LANES)], lvec16, mask=m)
            plsc.store_compressed(cpos.at[pl.ds(ccnt, LANES)], pos16, mask=m)
            return ccnt + _pc(m)

        titers = (app + LANES - 1) >> 4
        ccnt = lax.fori_loop(0, titers, ext, jnp.asarray(0, _i32))

        def group(g, carry):
            gbase = g * LANES
            lvec = cloc[pl.ds(gbase, LANES)]
            pvec = cpos[pl.ds(gbase, LANES)]
            gvalid = iota < (ccnt - gbase)
            gcnt = jnp.minimum(LANES, ccnt - gbase)
            rvec = plsc.load_gather(r_v, [pvec], mask=gvalid)
            iota_d = iota * DIM

            def dloop(d, carry2):
                dv = jnp.full((LANES,), 0, _i32) + d
                ent = gather_fn(dv, lvec, gvalid)
                rel = plsc.load_gather(rel64, [dv, rvec], mask=gvalid)
                plsc.store_scatter(rg, [iota_d + d], ent + rel,
                                   mask=gvalid)
                return carry2

            lax.fori_loop(0, DIM, dloop, 0)

            for j in range(LANES):
                pj = pvec[j]

                @pl.when(j < gcnt)
                def _():
                    pltpu.make_async_copy(
                        rg.at[pl.ds(j * DIM, DIM)],
                        out_ref.at[pl.ds(pj * DIM, DIM)], osem).start()

            def drain(j, carry2):
                pltpu.make_async_copy(rg.at[pl.ds(0, DIM)],
                                      out_ref.at[pl.ds(0, DIM)], osem).wait()
                return carry2

            lax.fori_loop(0, gcnt, drain, 0)
            return carry

        ngr = (ccnt + LANES - 1) >> 4
        lax.fori_loop(0, ngr, group, 0)

    def chunk_member(ch):
        def fn(loc16):
            return (loc16 >> CH_SHIFT) == ch, loc16 & (CH - 1)
        return fn

    def tail_member(loc16):
        return (loc16 >> CH_SHIFT) == nch_full, loc16 & (CH - 1)

    def chunk_gather(par):
        parv = jnp.full((LANES,), 0, _i32) + par

        def fn(dv, lvec, gvalid):
            return plsc.load_gather(chunk3, [parv, dv, lvec], mask=gvalid)
        return fn

    def tail_gather(dv, lvec, gvalid):
        return plsc.load_gather(tail64, [dv, lvec], mask=gvalid)

    def sweep(app):
        @pl.when(nch_full > 0)
        def _():
            issue_chunk(jnp.asarray(0, _i32), 0, csem0)

        def pair(cc, carry):
            ch0 = cc * 2
            ch1 = ch0 + 1

            @pl.when(ch1 < nch_full)
            def _():
                issue_chunk(ch1, 1, csem1)

            wait_chunk(0, csem0)
            process_members(app, chunk_member(ch0), chunk_gather(0))

            @pl.when(ch1 < nch_full)
            def _():
                @pl.when(ch1 + 1 < nch_full)
                def _():
                    issue_chunk(ch1 + 1, 0, csem0)

                wait_chunk(1, csem1)
                process_members(app, chunk_member(ch1), chunk_gather(1))

            return carry

        lax.fori_loop(0, (nch_full + 1) >> 1, pair, 0)

        process_members(app, tail_member, tail_gather)

    total0, app0 = scan_round(jnp.asarray(0, _i32))
    sweep(app0)

    def more(carry):
        rnd, total = carry
        return (rnd * CAP) < total

    def round_body(carry):
        rnd, total = carry
        _, app = scan_round(rnd)
        sweep(app)
        return rnd + 1, total

    lax.while_loop(more, round_body, (jnp.asarray(1, _i32), total0))


@jax.jit
def _transe(e1_1d, r_1d, ent_t, rel_t, tail_t):
    mesh = plsc.VectorSubcoreMesh(core_axis_name="c", subcore_axis_name="s")
    kern = pl.kernel(
        _body,
        mesh=mesh,
        compiler_params=pltpu.CompilerParams(needs_layout_passes=False),
        out_type=jax.ShapeDtypeStruct((BATCH * DIM,), jnp.float32),
        scratch_types=[
            pltpu.VMEM((E1_PIECE,), _i32),
            pltpu.VMEM((BATCH,), _i32),
            pltpu.VMEM((LIST,), _i32),
            pltpu.VMEM((LIST,), _i32),
            pltpu.VMEM((LIST,), _i32),
            pltpu.VMEM((LIST,), _i32),
            pltpu.VMEM((2, DIM, CH), jnp.float32),
            pltpu.VMEM((DIM, NUM_REL), jnp.float32),
            pltpu.VMEM((DIM, TAILN), jnp.float32),
            pltpu.VMEM((LANES * DIM,), jnp.float32),
            pltpu.SemaphoreType.DMA,
            pltpu.SemaphoreType.DMA,
            pltpu.SemaphoreType.DMA,
            pltpu.SemaphoreType.DMA,
        ],
    )
    return kern(e1_1d, r_1d, ent_t, rel_t, tail_t)


def kernel(e1, r, entity_table, relation_table):
    out = _transe(e1, r, entity_table.T, relation_table.T,
                  entity_table[TAIL0:].T)
    return out.reshape(BATCH, DIM)


# speedup vs baseline: 1.0457x; 1.0418x over previous
import functools

import jax
import jax.numpy as jnp
from jax import lax
from jax.experimental import pallas as pl
from jax.experimental.pallas import tpu as pltpu
from jax.experimental.pallas import tpu_sc as plsc

NUM_CORES = 2
NUM_SUBCORES = 16
NUM_WORKERS = NUM_CORES * NUM_SUBCORES
LANES = 16

BATCH = 16384
DIM = 64
ENT = 1000000
NUM_REL = 1000

PART_SHIFT = 15
PART = 1 << PART_SHIFT
CH_SHIFT = 8
CH = 1 << CH_SHIFT
CAP = 1024
LIST = CAP + LANES
E1_PIECE = 2048
TAIL0 = (ENT // 128) * 128
TAILN = ENT - TAIL0

_i32 = jnp.int32


def _pc(mask):
    n = plsc.all_reduce_population_count(mask)
    n = jnp.asarray(n)
    return n[0] if n.ndim else n


def _body(e1_ref, r_ref, ent_t_ref, rel_t_ref, tail_t_ref, out_ref,
          e1buf, r_v, pids, ppos, cloc, cpos, chunk3, rel64, tail64, rg,
          csem0, csem1, osem, ssem):
    wid = lax.axis_index("s") * NUM_CORES + lax.axis_index("c")
    part_base = wid * PART
    span = jnp.maximum(0, jnp.minimum(PART, TAIL0 - part_base))
    nch_full = span >> CH_SHIFT

    iota = lax.iota(_i32, LANES)

    pltpu.sync_copy(r_ref, r_v)
    pltpu.sync_copy(rel_t_ref, rel64)
    pltpu.sync_copy(tail_t_ref, tail64)

    def scan_round(rnd):
        lo = rnd * CAP
        hi = lo + CAP

        def piece(p, carry):
            cnt, app = carry
            pltpu.sync_copy(e1_ref.at[pl.ds(p * E1_PIECE, E1_PIECE)], e1buf)

            def step(t, carry2):
                cnt2, app2 = carry2
                ev = e1buf[pl.ds(t * LANES, LANES)]
                m = (ev >> PART_SHIFT) == wid
                mi = m.astype(_i32)
                excl = plsc.cumsum(mi) - mi
                rank = cnt2 + excl
                sel = m & (rank >= lo) & (rank < hi)
                plsc.store_compressed(pids.at[pl.ds(app2, LANES)],
                                      ev - part_base, mask=sel)
                posv = iota + (p * E1_PIECE + t * LANES)
                plsc.store_compressed(ppos.at[pl.ds(app2, LANES)], posv,
                                      mask=sel)
                return cnt2 + _pc(m), app2 + _pc(sel)

            return lax.fori_loop(0, E1_PIECE // LANES, step, (cnt, app))

        cnt, app = lax.fori_loop(0, BATCH // E1_PIECE, piece,
                                 (jnp.asarray(0, _i32), jnp.asarray(0, _i32)))
        pids[pl.ds(app, LANES)] = jnp.full((LANES,), -1, _i32)
        return cnt, app

    def issue_chunk(ch, par_buf, sem):
        col0 = pl.multiple_of(part_base + ch * CH, CH)
        pltpu.make_async_copy(ent_t_ref.at[:, pl.ds(col0, CH)],
                              chunk3.at[par_buf], sem).start()

    def wait_chunk(par_buf, sem):
        pltpu.make_async_copy(ent_t_ref.at[:, pl.ds(0, CH)],
                              chunk3.at[par_buf], sem).wait()

    def process_members(app, member_fn, gather_fn):
        def ext(t, ccnt):
            base = t * LANES
            loc16 = pids[pl.ds(base, LANES)]
            pos16 = ppos[pl.ds(base, LANES)]
            m, lvec16 = member_fn(loc16)
            plsc.store_compressed(cloc.at[pl.ds(ccnt, LANES)], lvec16, mask=m)
            plsc.store_compressed(cpos.at[pl.ds(ccnt, LANES)], pos16, mask=m)
            return ccnt + _pc(m)

        titers = (app + LANES - 1) >> 4
        ccnt = lax.fori_loop(0, titers, ext, jnp.asarray(0, _i32))

        mrep = iota >> 2
        d4 = iota & 3

        def pack(g, carry):
            gbase = g * 4
            sel = gbase + mrep
            pm = sel < ccnt
            lvec = plsc.load_gather(cloc, [sel], mask=pm)
            pvec = plsc.load_gather(cpos, [sel], mask=pm)
            rvec = plsc.load_gather(r_v, [pvec], mask=pm)
            rg_idx = mrep * DIM + d4

            def dloop(dd, carry2):
                dvec = d4 + dd * 4
                ent = gather_fn(dvec, lvec, pm)
                rel = plsc.load_gather(rel64, [dvec, rvec], mask=pm)
                plsc.store_scatter(rg, [rg_idx + dd * 4], ent + rel,
                                   mask=pm)
                return carry2

            lax.fori_loop(0, DIM // 4, dloop, 0)

            pcnt = jnp.minimum(4, ccnt - gbase)
            for j in range(4):
                pj = pvec[j * 4]

                @pl.when(j < pcnt)
                def _():
                    pltpu.make_async_copy(
                        rg.at[pl.ds(j * DIM, DIM)],
                        out_ref.at[pl.ds(pj * DIM, DIM)], osem).start()

            def drain(j, carry2):
                pltpu.make_async_copy(rg.at[pl.ds(0, DIM)],
                                      out_ref.at[pl.ds(0, DIM)], osem).wait()
                return carry2

            lax.fori_loop(0, pcnt, drain, 0)
            return carry

        npk = (ccnt + 3) >> 2
        lax.fori_loop(0, npk, pack, 0)

    def chunk_member(ch):
        def fn(loc16):
            return (loc16 >> CH_SHIFT) == ch, loc16 & (CH - 1)
        return fn

    def tail_member(loc16):
        return (loc16 >> CH_SHIFT) == nch_full, loc16 & (CH - 1)

    def chunk_gather(par):
        parv = jnp.full((LANES,), 0, _i32) + par

        def fn(dv, lvec, gvalid):
            return plsc.load_gather(chunk3, [parv, dv, lvec], mask=gvalid)
        return fn

    def tail_gather(dv, lvec, gvalid):
        return plsc.load_gather(tail64, [dv, lvec], mask=gvalid)

    def sweep(app):
        @pl.when(nch_full > 0)
        def _():
            issue_chunk(jnp.asarray(0, _i32), 0, csem0)

        def pair(cc, carry):
            ch0 = cc * 2
            ch1 = ch0 + 1

            @pl.when(ch1 < nch_full)
            def _():
                issue_chunk(ch1, 1, csem1)

            wait_chunk(0, csem0)
            process_members(app, chunk_member(ch0), chunk_gather(0))

            @pl.when(ch1 < nch_full)
            def _():
                @pl.when(ch1 + 1 < nch_full)
                def _():
                    issue_chunk(ch1 + 1, 0, csem0)

                wait_chunk(1, csem1)
                process_members(app, chunk_member(ch1), chunk_gather(1))

            return carry

        lax.fori_loop(0, (nch_full + 1) >> 1, pair, 0)

        process_members(app, tail_member, tail_gather)

    total0, app0 = scan_round(jnp.asarray(0, _i32))
    sweep(app0)

    def more(carry):
        rnd, total = carry
        return (rnd * CAP) < total

    def round_body(carry):
        rnd, total = carry
        _, app = scan_round(rnd)
        sweep(app)
        return rnd + 1, total

    lax.while_loop(more, round_body, (jnp.asarray(1, _i32), total0))


@jax.jit
def _transe(e1_1d, r_1d, ent_t, rel_t, tail_t):
    mesh = plsc.VectorSubcoreMesh(core_axis_name="c", subcore_axis_name="s")
    kern = pl.kernel(
        _body,
        mesh=mesh,
        compiler_params=pltpu.CompilerParams(needs_layout_passes=False),
        out_type=jax.ShapeDtypeStruct((BATCH * DIM,), jnp.float32),
        scratch_types=[
            pltpu.VMEM((E1_PIECE,), _i32),
            pltpu.VMEM((BATCH,), _i32),
            pltpu.VMEM((LIST,), _i32),
            pltpu.VMEM((LIST,), _i32),
            pltpu.VMEM((LIST,), _i32),
            pltpu.VMEM((LIST,), _i32),
            pltpu.VMEM((2, DIM, CH), jnp.float32),
            pltpu.VMEM((DIM, NUM_REL), jnp.float32),
            pltpu.VMEM((DIM, TAILN), jnp.float32),
            pltpu.VMEM((LANES * DIM,), jnp.float32),
            pltpu.SemaphoreType.DMA,
            pltpu.SemaphoreType.DMA,
            pltpu.SemaphoreType.DMA,
            pltpu.SemaphoreType.DMA,
        ],
    )
    return kern(e1_1d, r_1d, ent_t, rel_t, tail_t)


def kernel(e1, r, entity_table, relation_table):
    out = _transe(e1, r, entity_table.T, relation_table.T,
                  entity_table[TAIL0:].T)
    return out.reshape(BATCH, DIM)


# speedup vs baseline: 1.0717x; 1.0249x over previous
import functools

import jax
import jax.numpy as jnp
from jax import lax
from jax.experimental import pallas as pl
from jax.experimental.pallas import tpu as pltpu
from jax.experimental.pallas import tpu_sc as plsc

NUM_CORES = 2
NUM_SUBCORES = 16
NUM_WORKERS = NUM_CORES * NUM_SUBCORES
LANES = 16

BATCH = 16384
DIM = 64
ENT = 1000000
NUM_REL = 1000

PART_SHIFT = 15
PART = 1 << PART_SHIFT
CH_SHIFT = 8
CH = 1 << CH_SHIFT
CAP = 1024
LIST = CAP + LANES
E1_PIECE = 2048
TAIL0 = (ENT // 128) * 128
TAILN = ENT - TAIL0

_i32 = jnp.int32


def _pc(mask):
    n = plsc.all_reduce_population_count(mask)
    n = jnp.asarray(n)
    return n[0] if n.ndim else n


def _body(e1_ref, r_ref, ent_t_ref, rel_t_ref, tail_t_ref, out_ref,
          e1buf, r_v, pids, ppos, cloc, cpos, chunk3, rel64, tail64, rg,
          csem0, csem1, osem, ssem):
    wid = lax.axis_index("s") * NUM_CORES + lax.axis_index("c")
    nch_all = TAIL0 >> CH_SHIFT
    cstart = (wid * nch_all) >> 5
    cend = ((wid + 1) * nch_all) >> 5
    cend_m = cend + (wid == NUM_WORKERS - 1).astype(_i32)
    part_base = cstart << CH_SHIFT
    nch_full = cend - cstart

    iota = lax.iota(_i32, LANES)

    pltpu.sync_copy(r_ref, r_v)
    pltpu.sync_copy(rel_t_ref, rel64)
    pltpu.sync_copy(tail_t_ref, tail64)

    def scan_round(rnd):
        lo = rnd * CAP
        hi = lo + CAP

        def piece(p, carry):
            cnt, app = carry
            pltpu.sync_copy(e1_ref.at[pl.ds(p * E1_PIECE, E1_PIECE)], e1buf)

            def step(t, carry2):
                cnt2, app2 = carry2
                ev = e1buf[pl.ds(t * LANES, LANES)]
                cg = ev >> CH_SHIFT
                m = (cg >= cstart) & (cg < cend_m)
                mi = m.astype(_i32)
                excl = plsc.cumsum(mi) - mi
                rank = cnt2 + excl
                sel = m & (rank >= lo) & (rank < hi)
                plsc.store_compressed(pids.at[pl.ds(app2, LANES)],
                                      ev - part_base, mask=sel)
                posv = iota + (p * E1_PIECE + t * LANES)
                plsc.store_compressed(ppos.at[pl.ds(app2, LANES)], posv,
                                      mask=sel)
                return cnt2 + _pc(m), app2 + _pc(sel)

            return lax.fori_loop(0, E1_PIECE // LANES, step, (cnt, app))

        cnt, app = lax.fori_loop(0, BATCH // E1_PIECE, piece,
                                 (jnp.asarray(0, _i32), jnp.asarray(0, _i32)))
        pids[pl.ds(app, LANES)] = jnp.full((LANES,), -1, _i32)
        return cnt, app

    def issue_chunk(ch, par_buf, sem):
        col0 = pl.multiple_of(part_base + ch * CH, CH)
        pltpu.make_async_copy(ent_t_ref.at[:, pl.ds(col0, CH)],
                              chunk3.at[par_buf], sem).start()

    def wait_chunk(par_buf, sem):
        pltpu.make_async_copy(ent_t_ref.at[:, pl.ds(0, CH)],
                              chunk3.at[par_buf], sem).wait()

    def process_members(app, member_fn, gather_fn):
        def ext(t, ccnt):
            base = t * LANES
            loc16 = pids[pl.ds(base, LANES)]
            pos16 = ppos[pl.ds(base, LANES)]
            m, lvec16 = member_fn(loc16)
            plsc.store_compressed(cloc.at[pl.ds(ccnt, LANES)], lvec16, mask=m)
            plsc.store_compressed(cpos.at[pl.ds(ccnt, LANES)], pos16, mask=m)
            return ccnt + _pc(m)

        titers = (app + LANES - 1) >> 4
        ccnt = lax.fori_loop(0, titers, ext, jnp.asarray(0, _i32))

        mrep = iota >> 2
        d4 = iota & 3

        def pack(g, carry):
            gbase = g * 4
            sel = gbase + mrep
            pm = sel < ccnt
            lvec = plsc.load_gather(cloc, [sel], mask=pm)
            pvec = plsc.load_gather(cpos, [sel], mask=pm)
            rvec = plsc.load_gather(r_v, [pvec], mask=pm)
            rg_idx = mrep * DIM + d4

            def dloop(dd, carry2):
                dvec = d4 + dd * 4
                ent = gather_fn(dvec, lvec, pm)
                rel = plsc.load_gather(rel64, [dvec, rvec], mask=pm)
                plsc.store_scatter(rg, [rg_idx + dd * 4], ent + rel,
                                   mask=pm)
                return carry2

            lax.fori_loop(0, DIM // 4, dloop, 0)

            pcnt = jnp.minimum(4, ccnt - gbase)
            for j in range(4):
                pj = pvec[j * 4]

                @pl.when(j < pcnt)
                def _():
                    pltpu.make_async_copy(
                        rg.at[pl.ds(j * DIM, DIM)],
                        out_ref.at[pl.ds(pj * DIM, DIM)], osem).start()

            def drain(j, carry2):
                pltpu.make_async_copy(rg.at[pl.ds(0, DIM)],
                                      out_ref.at[pl.ds(0, DIM)], osem).wait()
                return carry2

            lax.fori_loop(0, pcnt, drain, 0)
            return carry

        npk = (ccnt + 3) >> 2
        lax.fori_loop(0, npk, pack, 0)

    def chunk_member(ch):
        def fn(loc16):
            return (loc16 >> CH_SHIFT) == ch, loc16 & (CH - 1)
        return fn

    def tail_member(loc16):
        return (loc16 >> CH_SHIFT) == nch_full, loc16 & (CH - 1)

    def chunk_gather(par):
        parv = jnp.full((LANES,), 0, _i32) + par

        def fn(dv, lvec, gvalid):
            return plsc.load_gather(chunk3, [parv, dv, lvec], mask=gvalid)
        return fn

    def tail_gather(dv, lvec, gvalid):
        return plsc.load_gather(tail64, [dv, lvec], mask=gvalid)

    def sweep(app):
        @pl.when(nch_full > 0)
        def _():
            issue_chunk(jnp.asarray(0, _i32), 0, csem0)

        def pair(cc, carry):
            ch0 = cc * 2
            ch1 = ch0 + 1

            @pl.when(ch1 < nch_full)
            def _():
                issue_chunk(ch1, 1, csem1)

            wait_chunk(0, csem0)
            process_members(app, chunk_member(ch0), chunk_gather(0))

            @pl.when(ch1 < nch_full)
            def _():
                @pl.when(ch1 + 1 < nch_full)
                def _():
                    issue_chunk(ch1 + 1, 0, csem0)

                wait_chunk(1, csem1)
                process_members(app, chunk_member(ch1), chunk_gather(1))

            return carry

        lax.fori_loop(0, (nch_full + 1) >> 1, pair, 0)

        process_members(app, tail_member, tail_gather)

    total0, app0 = scan_round(jnp.asarray(0, _i32))
    sweep(app0)

    def more(carry):
        rnd, total = carry
        return (rnd * CAP) < total

    def round_body(carry):
        rnd, total = carry
        _, app = scan_round(rnd)
        sweep(app)
        return rnd + 1, total

    lax.while_loop(more, round_body, (jnp.asarray(1, _i32), total0))


@jax.jit
def _transe(e1_1d, r_1d, ent_t, rel_t, tail_t):
    mesh = plsc.VectorSubcoreMesh(core_axis_name="c", subcore_axis_name="s")
    kern = pl.kernel(
        _body,
        mesh=mesh,
        compiler_params=pltpu.CompilerParams(needs_layout_passes=False),
        out_type=jax.ShapeDtypeStruct((BATCH * DIM,), jnp.float32),
        scratch_types=[
            pltpu.VMEM((E1_PIECE,), _i32),
            pltpu.VMEM((BATCH,), _i32),
            pltpu.VMEM((LIST,), _i32),
            pltpu.VMEM((LIST,), _i32),
            pltpu.VMEM((LIST,), _i32),
            pltpu.VMEM((LIST,), _i32),
            pltpu.VMEM((2, DIM, CH), jnp.float32),
            pltpu.VMEM((DIM, NUM_REL), jnp.float32),
            pltpu.VMEM((DIM, TAILN), jnp.float32),
            pltpu.VMEM((LANES * DIM,), jnp.float32),
            pltpu.SemaphoreType.DMA,
            pltpu.SemaphoreType.DMA,
            pltpu.SemaphoreType.DMA,
            pltpu.SemaphoreType.DMA,
        ],
    )
    return kern(e1_1d, r_1d, ent_t, rel_t, tail_t)


def kernel(e1, r, entity_table, relation_table):
    out = _transe(e1, r, entity_table.T, relation_table.T,
                  entity_table[TAIL0:].T)
    return out.reshape(BATCH, DIM)


# speedup vs baseline: 1.1084x; 1.0342x over previous
import functools

import jax
import jax.numpy as jnp
from jax import lax
from jax.experimental import pallas as pl
from jax.experimental.pallas import tpu as pltpu
from jax.experimental.pallas import tpu_sc as plsc

NUM_CORES = 2
NUM_SUBCORES = 16
NUM_WORKERS = NUM_CORES * NUM_SUBCORES
LANES = 16

BATCH = 16384
DIM = 64
ENT = 1000000
NUM_REL = 1000

PART_SHIFT = 15
PART = 1 << PART_SHIFT
CH_SHIFT = 8
CH = 1 << CH_SHIFT
CAP = 1024
LIST = CAP + LANES
E1_PIECE = 2048
TAIL0 = (ENT // 128) * 128
TAILN = ENT - TAIL0

_i32 = jnp.int32


def _pc(mask):
    n = plsc.all_reduce_population_count(mask)
    n = jnp.asarray(n)
    return n[0] if n.ndim else n


def _body(e1_ref, r_ref, ent_t_ref, rel_t_ref, tail_t_ref, out_ref,
          e1buf, r_v, pids, ppos, cloc, cpos, chunk3, rel64, tail64, rg,
          csem0, csem1, osem, ssem):
    wid = lax.axis_index("s") * NUM_CORES + lax.axis_index("c")
    nch_all = TAIL0 >> CH_SHIFT
    cstart = (wid * nch_all) >> 5
    cend = ((wid + 1) * nch_all) >> 5
    cend_m = cend + (wid == NUM_WORKERS - 1).astype(_i32)
    part_base = cstart << CH_SHIFT
    nch_full = cend - cstart

    iota = lax.iota(_i32, LANES)

    stage = [pltpu.make_async_copy(r_ref, r_v, ssem),
             pltpu.make_async_copy(rel_t_ref, rel64, ssem),
             pltpu.make_async_copy(tail_t_ref, tail64, ssem)]
    for cp in stage:
        cp.start()

    def scan_round(rnd):
        lo = rnd * CAP
        hi = lo + CAP

        def piece(p, carry):
            cnt, app = carry
            pltpu.sync_copy(e1_ref.at[pl.ds(p * E1_PIECE, E1_PIECE)], e1buf)

            def step(t, carry2):
                cnt2, app2 = carry2
                ev = e1buf[pl.ds(t * LANES, LANES)]
                cg = ev >> CH_SHIFT
                m = (cg >= cstart) & (cg < cend_m)
                mi = m.astype(_i32)
                excl = plsc.cumsum(mi) - mi
                rank = cnt2 + excl
                sel = m & (rank >= lo) & (rank < hi)
                plsc.store_compressed(pids.at[pl.ds(app2, LANES)],
                                      ev - part_base, mask=sel)
                posv = iota + (p * E1_PIECE + t * LANES)
                plsc.store_compressed(ppos.at[pl.ds(app2, LANES)], posv,
                                      mask=sel)
                return cnt2 + _pc(m), app2 + _pc(sel)

            return lax.fori_loop(0, E1_PIECE // LANES, step, (cnt, app))

        cnt, app = lax.fori_loop(0, BATCH // E1_PIECE, piece,
                                 (jnp.asarray(0, _i32), jnp.asarray(0, _i32)))
        pids[pl.ds(app, LANES)] = jnp.full((LANES,), -1, _i32)
        return cnt, app

    def issue_chunk(ch, par_buf, sem):
        col0 = pl.multiple_of(part_base + ch * CH, CH)
        pltpu.make_async_copy(ent_t_ref.at[:, pl.ds(col0, CH)],
                              chunk3.at[par_buf], sem).start()

    def wait_chunk(par_buf, sem):
        pltpu.make_async_copy(ent_t_ref.at[:, pl.ds(0, CH)],
                              chunk3.at[par_buf], sem).wait()

    def process_members(app, member_fn, gather_fn):
        def ext(t, ccnt):
            base = t * LANES
            loc16 = pids[pl.ds(base, LANES)]
            pos16 = ppos[pl.ds(base, LANES)]
            m, lvec16 = member_fn(loc16)
            plsc.store_compressed(cloc.at[pl.ds(ccnt, LANES)], lvec16, mask=m)
            plsc.store_compressed(cpos.at[pl.ds(ccnt, LANES)], pos16, mask=m)
            return ccnt + _pc(m)

        titers = (app + LANES - 1) >> 4
        ccnt = lax.fori_loop(0, titers, ext, jnp.asarray(0, _i32))

        mrep = iota >> 2
        d4 = iota & 3

        def pack(g, carry):
            gbase = g * 4
            sel = gbase + mrep
            pm = sel < ccnt
            lvec = plsc.load_gather(cloc, [sel], mask=pm)
            pvec = plsc.load_gather(cpos, [sel], mask=pm)
            rvec = plsc.load_gather(r_v, [pvec], mask=pm)
            rg_idx = mrep * DIM + d4

            def dloop(dd, carry2):
                dvec = d4 + dd * 4
                ent = gather_fn(dvec, lvec, pm)
                rel = plsc.load_gather(rel64, [dvec, rvec], mask=pm)
                plsc.store_scatter(rg, [rg_idx + dd * 4], ent + rel,
                                   mask=pm)
                return carry2

            lax.fori_loop(0, DIM // 4, dloop, 0)

            pcnt = jnp.minimum(4, ccnt - gbase)
            for j in range(4):
                pj = pvec[j * 4]

                @pl.when(j < pcnt)
                def _():
                    pltpu.make_async_copy(
                        rg.at[pl.ds(j * DIM, DIM)],
                        out_ref.at[pl.ds(pj * DIM, DIM)], osem).start()

            def drain(j, carry2):
                pltpu.make_async_copy(rg.at[pl.ds(0, DIM)],
                                      out_ref.at[pl.ds(0, DIM)], osem).wait()
                return carry2

            lax.fori_loop(0, pcnt, drain, 0)
            return carry

        npk = (ccnt + 3) >> 2
        lax.fori_loop(0, npk, pack, 0)

    def chunk_member(ch):
        def fn(loc16):
            return (loc16 >> CH_SHIFT) == ch, loc16 & (CH - 1)
        return fn

    def tail_member(loc16):
        return (loc16 >> CH_SHIFT) == nch_full, loc16 & (CH - 1)

    def chunk_gather(par):
        parv = jnp.full((LANES,), 0, _i32) + par

        def fn(dv, lvec, gvalid):
            return plsc.load_gather(chunk3, [parv, dv, lvec], mask=gvalid)
        return fn

    def tail_gather(dv, lvec, gvalid):
        return plsc.load_gather(tail64, [dv, lvec], mask=gvalid)

    def sweep(app, primed=False):
        if not primed:
            @pl.when(nch_full > 0)
            def _():
                issue_chunk(jnp.asarray(0, _i32), 0, csem0)

        def pair(cc, carry):
            ch0 = cc * 2
            ch1 = ch0 + 1

            @pl.when(ch1 < nch_full)
            def _():
                issue_chunk(ch1, 1, csem1)

            wait_chunk(0, csem0)
            process_members(app, chunk_member(ch0), chunk_gather(0))

            @pl.when(ch1 < nch_full)
            def _():
                @pl.when(ch1 + 1 < nch_full)
                def _():
                    issue_chunk(ch1 + 1, 0, csem0)

                wait_chunk(1, csem1)
                process_members(app, chunk_member(ch1), chunk_gather(1))

            return carry

        lax.fori_loop(0, (nch_full + 1) >> 1, pair, 0)

        process_members(app, tail_member, tail_gather)

    @pl.when(nch_full > 0)
    def _():
        issue_chunk(jnp.asarray(0, _i32), 0, csem0)

    total0, app0 = scan_round(jnp.asarray(0, _i32))
    for cp in stage:
        cp.wait()
    sweep(app0, primed=True)

    def more(carry):
        rnd, total = carry
        return (rnd * CAP) < total

    def round_body(carry):
        rnd, total = carry
        _, app = scan_round(rnd)
        sweep(app)
        return rnd + 1, total

    lax.while_loop(more, round_body, (jnp.asarray(1, _i32), total0))


@jax.jit
def _transe(e1_1d, r_1d, ent_t, rel_t, tail_t):
    mesh = plsc.VectorSubcoreMesh(core_axis_name="c", subcore_axis_name="s")
    kern = pl.kernel(
        _body,
        mesh=mesh,
        compiler_params=pltpu.CompilerParams(needs_layout_passes=False),
        out_type=jax.ShapeDtypeStruct((BATCH * DIM,), jnp.float32),
        scratch_types=[
            pltpu.VMEM((E1_PIECE,), _i32),
            pltpu.VMEM((BATCH,), _i32),
            pltpu.VMEM((LIST,), _i32),
            pltpu.VMEM((LIST,), _i32),
            pltpu.VMEM((LIST,), _i32),
            pltpu.VMEM((LIST,), _i32),
            pltpu.VMEM((2, DIM, CH), jnp.float32),
            pltpu.VMEM((DIM, NUM_REL), jnp.float32),
            pltpu.VMEM((DIM, TAILN), jnp.float32),
            pltpu.VMEM((LANES * DIM,), jnp.float32),
            pltpu.SemaphoreType.DMA,
            pltpu.SemaphoreType.DMA,
            pltpu.SemaphoreType.DMA,
            pltpu.SemaphoreType.DMA,
        ],
    )
    return kern(e1_1d, r_1d, ent_t, rel_t, tail_t)


def kernel(e1, r, entity_table, relation_table):
    out = _transe(e1, r, entity_table.T, relation_table.T,
                  entity_table[TAIL0:].T)
    return out.reshape(BATCH, DIM)
